# Initial kernel scaffold; baseline (speedup 1.0000x reference)
import jax
import jax.numpy as jnp
from jax.experimental import pallas as pl


def kernel(x, edge_index, W_in, b_in, W0, b0, ls0, lb0, W1, b1, ls1, lb1, W2, b2, ls2, lb2):
    raise NotImplementedError("write your pallas kernel here")


# speedup vs baseline: 4.6168x; 4.6168x over previous
import functools

import jax
import jax.numpy as jnp
from jax import lax
from jax.experimental import pallas as pl
from jax.experimental.pallas import tpu as pltpu
from jax.experimental.pallas import tpu_sc as plsc

N = 10000
E = 320000
D = 128
NC = 2
NS = 16
K = 128
CHUNKS = 160
EPT = CHUNKS * K
EPAD = NS * EPT
NPAD = 10240
HALF = NPAD // NC
ACC_R = 5248
DUMP = 5184
DEGW = 16
DRPT = NPAD // NS
ARPT = ACC_R // NS
ORPT = HALF // NS
RB = 1000


def _mesh():
    return plsc.VectorSubcoreMesh(core_axis_name="c", subcore_axis_name="s")


def _sc_degree(dst3):
    HC = CHUNKS // 2

    @functools.partial(
        pl.kernel,
        out_type=jax.ShapeDtypeStruct((NC, NPAD, DEGW), jnp.float32),
        mesh=_mesh(),
        scratch_types=[
            pltpu.VMEM((HC, K), jnp.int32),
            pltpu.VMEM((K, DEGW), jnp.float32),
            pltpu.VMEM((K, DEGW), jnp.float32),
            pltpu.VMEM_SHARED((NPAD, DEGW), jnp.float32),
        ],
    )
    def deg_kernel(dst_hbm, out_hbm, dst_v, ones_v, zero_v, acc):
        c = lax.axis_index("c")
        s = lax.axis_index("s")
        ones16 = jnp.ones((16,), jnp.float32)
        zeros16 = jnp.zeros((16,), jnp.float32)

        def fill(i, carry):
            ones_v[i, pl.ds(0, 16)] = ones16
            zero_v[i, pl.ds(0, 16)] = zeros16
            return carry

        lax.fori_loop(0, K, fill, 0)
        for m in range(DRPT // K):
            pltpu.sync_copy(zero_v, acc.at[pl.ds(s * DRPT + m * K, K)])
        pltpu.sync_copy(dst_hbm.at[s].at[pl.ds(c * HC, HC)], dst_v)
        plsc.subcore_barrier()

        def chunk(j, carry):
            pltpu.sync_copy(ones_v, acc.at[dst_v.at[j]], add=True)
            return carry

        lax.fori_loop(0, HC, chunk, 0)
        plsc.subcore_barrier()
        pltpu.sync_copy(acc.at[pl.ds(s * DRPT, DRPT)],
                        out_hbm.at[c].at[pl.ds(s * DRPT, DRPT)])

    return deg_kernel(dst3)


def _sc_scatter(src3, dst3, tp):
    @functools.partial(
        pl.kernel,
        out_type=jax.ShapeDtypeStruct((NC, HALF, D), jnp.float32),
        mesh=_mesh(),
        scratch_types=[
            pltpu.VMEM((CHUNKS, K), jnp.int32),
            pltpu.VMEM((CHUNKS, K), jnp.int32),
            pltpu.VMEM((K, D), jnp.float32),
            pltpu.VMEM((K, D), jnp.float32),
            pltpu.SemaphoreType.DMA,
            pltpu.VMEM_SHARED((ACC_R, D), jnp.float32),
        ],
    )
    def scat_kernel(src_hbm, dst_hbm, tp_hbm, out_hbm,
                    src_v, dst_v, buf, zbuf, sem, acc):
        c = lax.axis_index("c")
        s = lax.axis_index("s")
        zeros16 = jnp.zeros((16,), jnp.float32)

        def fill(i, carry):
            for kk in range(D // 16):
                zbuf[i, pl.ds(kk * 16, 16)] = zeros16
            return carry

        lax.fori_loop(0, K, fill, 0)
        base = s * ARPT
        for m in range(ARPT // K):
            pltpu.sync_copy(zbuf, acc.at[pl.ds(base + m * K, K)])
        rem = ARPT % K
        if rem:
            pltpu.sync_copy(zbuf.at[pl.ds(0, rem)],
                            acc.at[pl.ds(base + (ARPT // K) * K, rem)])
        pltpu.sync_copy(src_hbm.at[s], src_v)
        pltpu.sync_copy(dst_hbm.at[s], dst_v)

        lo_bound = c * HALF

        def remap(j, carry):
            for kk in range(K // 16):
                idx = dst_v[j, pl.ds(kk * 16, 16)]
                loc = idx - lo_bound
                inr = jnp.logical_and(loc >= 0, loc < HALF)
                dump = jnp.full((16,), DUMP, jnp.int32)
                dst_v[j, pl.ds(kk * 16, 16)] = jnp.where(inr, loc, dump)
            return carry

        lax.fori_loop(0, CHUNKS, remap, 0)
        plsc.subcore_barrier()

        def chunk(j, carry):
            pltpu.async_copy(tp_hbm.at[src_v.at[j]], buf, sem).wait()
            pltpu.sync_copy(buf, acc.at[dst_v.at[j]], add=True)
            return carry

        lax.fori_loop(0, CHUNKS, chunk, 0)
        plsc.subcore_barrier()
        pltpu.sync_copy(acc.at[pl.ds(s * ORPT, ORPT)],
                        out_hbm.at[c].at[pl.ds(s * ORPT, ORPT)])

    return scat_kernel(src3, dst3, tp)


def _tc_dinv(degp):
    def body(p_ref, o_ref):
        p = p_ref[...]
        o_ref[...] = lax.rsqrt(1.0 + p[0] + p[1])

    return pl.pallas_call(
        body, out_shape=jax.ShapeDtypeStruct((NPAD, DEGW), jnp.float32)
    )(degp)


_DN = (((1,), (1,)), ((), ()))


def _tc_init(x, w_in, b_in2, w0, dinv):
    def body(x_ref, wi_ref, bi_ref, w0_ref, dv_ref, h_ref, tp_ref):
        h = lax.dot_general(x_ref[...], wi_ref[...], _DN,
                            preferred_element_type=jnp.float32) + bi_ref[...]
        h_ref[...] = h
        t = lax.dot_general(h, w0_ref[...], _DN,
                            preferred_element_type=jnp.float32)
        tp_ref[...] = t * dv_ref[...]

    bs_row = pl.BlockSpec((RB, D), lambda i: (i, 0))
    bs_w = pl.BlockSpec((D, D), lambda i: (0, 0))
    bs_b = pl.BlockSpec((1, D), lambda i: (0, 0))
    bs_dv = pl.BlockSpec((RB, 1), lambda i: (i, 0))
    return pl.pallas_call(
        body,
        grid=(N // RB,),
        in_specs=[bs_row, bs_w, bs_b, bs_w, bs_dv],
        out_specs=[bs_row, bs_row],
        out_shape=[jax.ShapeDtypeStruct((N, D), jnp.float32)] * 2,
    )(x, w_in, b_in2, w0, dinv)


def _layer_math(h, s_rows, tp, dv, b, ls_, lb_):
    agg = dv * (s_rows + tp) + b
    hn = jnp.where(agg > 0, agg, jnp.exp(agg) - 1.0)
    r = hn + h
    mu = jnp.mean(r, axis=-1, keepdims=True)
    dd = r - mu
    var = jnp.mean(dd * dd, axis=-1, keepdims=True)
    return dd * lax.rsqrt(var + 1e-5) * ls_ + lb_


def _tc_layer(h, s_rows, tp, dinv, b2, ls2_, lb2_, w_next):
    def body(h_ref, s_ref, tp_ref, dv_ref, b_ref, ls_ref, lb_ref, wn_ref,
             ho_ref, tpn_ref):
        dv = dv_ref[...]
        y = _layer_math(h_ref[...], s_ref[...], tp_ref[...], dv,
                        b_ref[...], ls_ref[...], lb_ref[...])
        ho_ref[...] = y
        tpn_ref[...] = lax.dot_general(y, wn_ref[...], _DN,
                                       preferred_element_type=jnp.float32) * dv

    bs_row = pl.BlockSpec((RB, D), lambda i: (i, 0))
    bs_w = pl.BlockSpec((D, D), lambda i: (0, 0))
    bs_b = pl.BlockSpec((1, D), lambda i: (0, 0))
    bs_dv = pl.BlockSpec((RB, 1), lambda i: (i, 0))
    return pl.pallas_call(
        body,
        grid=(N // RB,),
        in_specs=[bs_row, bs_row, bs_row, bs_dv, bs_b, bs_b, bs_b, bs_w],
        out_specs=[bs_row, bs_row],
        out_shape=[jax.ShapeDtypeStruct((N, D), jnp.float32)] * 2,
    )(h, s_rows, tp, dinv, b2, ls2_, lb2_, w_next)


def _tc_final(h, s_rows, tp, dinv, b2, ls2_, lb2_):
    def body(h_ref, s_ref, tp_ref, dv_ref, b_ref, ls_ref, lb_ref, ho_ref):
        ho_ref[...] = _layer_math(h_ref[...], s_ref[...], tp_ref[...],
                                  dv_ref[...], b_ref[...], ls_ref[...],
                                  lb_ref[...])

    bs_row = pl.BlockSpec((RB, D), lambda i: (i, 0))
    bs_b = pl.BlockSpec((1, D), lambda i: (0, 0))
    bs_dv = pl.BlockSpec((RB, 1), lambda i: (i, 0))
    return pl.pallas_call(
        body,
        grid=(N // RB,),
        in_specs=[bs_row, bs_row, bs_row, bs_dv, bs_b, bs_b, bs_b],
        out_specs=bs_row,
        out_shape=jax.ShapeDtypeStruct((N, D), jnp.float32),
    )(h, s_rows, tp, dinv, b2, ls2_, lb2_)


def kernel(x, edge_index, W_in, b_in,
           W0, b0, ls0, lb0, W1, b1, ls1, lb1, W2, b2, ls2, lb2):
    src = edge_index[0]
    dst = edge_index[1]
    pad = EPAD - E
    srcp = jnp.concatenate([src, jnp.zeros((pad,), jnp.int32)])
    dstp = jnp.concatenate([dst, jnp.full((pad,), N, jnp.int32)])
    src3 = srcp.reshape(NS, CHUNKS, K)
    dst3 = dstp.reshape(NS, CHUNKS, K)

    degp = _sc_degree(dst3)
    dinv_full = _tc_dinv(degp)
    dinv = dinv_full[:N, 0:1]

    h, tp = _tc_init(x, W_in, b_in.reshape(1, D), W0, dinv)

    for (b, ls_, lb_, w_next) in ((b0, ls0, lb0, W1), (b1, ls1, lb1, W2)):
        s_rows = _sc_scatter(src3, dst3, tp).reshape(NPAD, D)
        h, tp = _tc_layer(h, s_rows, tp, dinv, b.reshape(1, D),
                          ls_.reshape(1, D), lb_.reshape(1, D), w_next)

    s_rows = _sc_scatter(src3, dst3, tp).reshape(NPAD, D)
    return _tc_final(h, s_rows, tp, dinv, b2.reshape(1, D),
                     ls2.reshape(1, D), lb2.reshape(1, D))


# speedup vs baseline: 4.7311x; 1.0247x over previous
import functools

import jax
import jax.numpy as jnp
from jax import lax
from jax.experimental import pallas as pl
from jax.experimental.pallas import tpu as pltpu
from jax.experimental.pallas import tpu_sc as plsc

N = 10000
E = 320000
D = 128
NC = 2
NS = 16
K = 128
CHUNKS = 160
EPT = CHUNKS * K
EPAD = NS * EPT
NPAD = 10240
HALF = NPAD // NC
ACC_R = 5248
DUMP = 5184
DEGW = 16
DRPT = NPAD // NS
ARPT = ACC_R // NS
ORPT = HALF // NS
RB = 1000


def _mesh():
    return plsc.VectorSubcoreMesh(core_axis_name="c", subcore_axis_name="s")


def _sc_degree(dst3):
    HC = CHUNKS // 2

    @functools.partial(
        pl.kernel,
        out_type=jax.ShapeDtypeStruct((NC, NPAD, DEGW), jnp.float32),
        mesh=_mesh(),
        scratch_types=[
            pltpu.VMEM((HC, K), jnp.int32),
            pltpu.VMEM((K, DEGW), jnp.float32),
            pltpu.VMEM((K, DEGW), jnp.float32),
            pltpu.VMEM_SHARED((NPAD, DEGW), jnp.float32),
        ],
    )
    def deg_kernel(dst_hbm, out_hbm, dst_v, ones_v, zero_v, acc):
        c = lax.axis_index("c")
        s = lax.axis_index("s")
        ones16 = jnp.ones((16,), jnp.float32)
        zeros16 = jnp.zeros((16,), jnp.float32)

        def fill(i, carry):
            ones_v[i, pl.ds(0, 16)] = ones16
            zero_v[i, pl.ds(0, 16)] = zeros16
            return carry

        lax.fori_loop(0, K, fill, 0)
        for m in range(DRPT // K):
            pltpu.sync_copy(zero_v, acc.at[pl.ds(s * DRPT + m * K, K)])
        pltpu.sync_copy(dst_hbm.at[s].at[pl.ds(c * HC, HC)], dst_v)
        plsc.subcore_barrier()

        def chunk(j, carry):
            pltpu.sync_copy(ones_v, acc.at[dst_v.at[j]], add=True)
            return carry

        lax.fori_loop(0, HC, chunk, 0)
        plsc.subcore_barrier()
        pltpu.sync_copy(acc.at[pl.ds(s * DRPT, DRPT)],
                        out_hbm.at[c].at[pl.ds(s * DRPT, DRPT)])

    return deg_kernel(dst3)


def _sc_scatter(src3, dst3, tp):
    @functools.partial(
        pl.kernel,
        out_type=jax.ShapeDtypeStruct((NC, HALF, D), jnp.float32),
        mesh=_mesh(),
        scratch_types=[
            pltpu.VMEM((CHUNKS, K), jnp.int32),
            pltpu.VMEM((CHUNKS, K), jnp.int32),
            pltpu.VMEM((K, D), jnp.float32),
            pltpu.VMEM((K, D), jnp.float32),
            pltpu.SemaphoreType.DMA,
            pltpu.VMEM_SHARED((ACC_R, D), jnp.float32),
        ],
    )
    def scat_kernel(src_hbm, dst_hbm, tp_hbm, out_hbm,
                    src_v, dst_v, b0, b1, ssem, acc):
        c = lax.axis_index("c")
        s = lax.axis_index("s")
        zeros16 = jnp.zeros((16,), jnp.float32)

        def fill(i, carry):
            for kk in range(D // 16):
                b0[i, pl.ds(kk * 16, 16)] = zeros16
            return carry

        lax.fori_loop(0, K, fill, 0)
        base = s * ARPT
        for m in range(ARPT // K):
            pltpu.sync_copy(b0, acc.at[pl.ds(base + m * K, K)])
        rem = ARPT % K
        if rem:
            pltpu.sync_copy(b0.at[pl.ds(0, rem)],
                            acc.at[pl.ds(base + (ARPT // K) * K, rem)])
        pltpu.sync_copy(src_hbm.at[s], src_v)
        pltpu.sync_copy(dst_hbm.at[s], dst_v)

        lo_bound = c * HALF

        def remap(j, carry):
            for kk in range(K // 16):
                idx = dst_v[j, pl.ds(kk * 16, 16)]
                loc = idx - lo_bound
                inr = jnp.logical_and(loc >= 0, loc < HALF)
                dump = jnp.full((16,), DUMP, jnp.int32)
                dst_v[j, pl.ds(kk * 16, 16)] = jnp.where(inr, loc, dump)
            return carry

        lax.fori_loop(0, CHUNKS, remap, 0)
        plsc.subcore_barrier()

        def sync_g(j, buf):
            pltpu.sync_copy(tp_hbm.at[src_v.at[j]], buf)

        def fire_s(j, buf):
            pltpu.async_copy(buf, acc.at[dst_v.at[j]], ssem, add=True)

        def drain_s(j, buf):
            pltpu.make_async_copy(buf, acc.at[dst_v.at[j]], ssem).wait()

        def step(i, carry):
            j = 2 * i

            @pl.when(i > 0)
            def _():
                drain_s(j - 2, b0)
                drain_s(j - 1, b1)

            sync_g(j, b0)
            fire_s(j, b0)
            sync_g(j + 1, b1)
            fire_s(j + 1, b1)
            return carry

        lax.fori_loop(0, CHUNKS // 2, step, 0)
        drain_s(CHUNKS - 2, b0)
        drain_s(CHUNKS - 1, b1)
        plsc.subcore_barrier()
        pltpu.sync_copy(acc.at[pl.ds(s * ORPT, ORPT)],
                        out_hbm.at[c].at[pl.ds(s * ORPT, ORPT)])

    return scat_kernel(src3, dst3, tp)


def _tc_dinv(degp):
    def body(p_ref, o_ref):
        p = p_ref[...]
        o_ref[...] = lax.rsqrt(1.0 + p[0] + p[1])

    return pl.pallas_call(
        body, out_shape=jax.ShapeDtypeStruct((NPAD, DEGW), jnp.float32)
    )(degp)


_DN = (((1,), (1,)), ((), ()))


def _tc_init(x, w_in, b_in2, w0, dinv):
    def body(x_ref, wi_ref, bi_ref, w0_ref, dv_ref, h_ref, tp_ref):
        h = lax.dot_general(x_ref[...], wi_ref[...], _DN,
                            preferred_element_type=jnp.float32) + bi_ref[...]
        h_ref[...] = h
        t = lax.dot_general(h, w0_ref[...], _DN,
                            preferred_element_type=jnp.float32)
        tp_ref[...] = t * dv_ref[...]

    bs_row = pl.BlockSpec((RB, D), lambda i: (i, 0))
    bs_w = pl.BlockSpec((D, D), lambda i: (0, 0))
    bs_b = pl.BlockSpec((1, D), lambda i: (0, 0))
    bs_dv = pl.BlockSpec((RB, 1), lambda i: (i, 0))
    return pl.pallas_call(
        body,
        grid=(N // RB,),
        in_specs=[bs_row, bs_w, bs_b, bs_w, bs_dv],
        out_specs=[bs_row, bs_row],
        out_shape=[jax.ShapeDtypeStruct((N, D), jnp.float32)] * 2,
    )(x, w_in, b_in2, w0, dinv)


def _layer_math(h, s_rows, tp, dv, b, ls_, lb_):
    agg = dv * (s_rows + tp) + b
    hn = jnp.where(agg > 0, agg, jnp.exp(agg) - 1.0)
    r = hn + h
    mu = jnp.mean(r, axis=-1, keepdims=True)
    dd = r - mu
    var = jnp.mean(dd * dd, axis=-1, keepdims=True)
    return dd * lax.rsqrt(var + 1e-5) * ls_ + lb_


def _tc_layer(h, s_rows, tp, dinv, b2, ls2_, lb2_, w_next):
    def body(h_ref, s_ref, tp_ref, dv_ref, b_ref, ls_ref, lb_ref, wn_ref,
             ho_ref, tpn_ref):
        dv = dv_ref[...]
        y = _layer_math(h_ref[...], s_ref[...], tp_ref[...], dv,
                        b_ref[...], ls_ref[...], lb_ref[...])
        ho_ref[...] = y
        tpn_ref[...] = lax.dot_general(y, wn_ref[...], _DN,
                                       preferred_element_type=jnp.float32) * dv

    bs_row = pl.BlockSpec((RB, D), lambda i: (i, 0))
    bs_w = pl.BlockSpec((D, D), lambda i: (0, 0))
    bs_b = pl.BlockSpec((1, D), lambda i: (0, 0))
    bs_dv = pl.BlockSpec((RB, 1), lambda i: (i, 0))
    return pl.pallas_call(
        body,
        grid=(N // RB,),
        in_specs=[bs_row, bs_row, bs_row, bs_dv, bs_b, bs_b, bs_b, bs_w],
        out_specs=[bs_row, bs_row],
        out_shape=[jax.ShapeDtypeStruct((N, D), jnp.float32)] * 2,
    )(h, s_rows, tp, dinv, b2, ls2_, lb2_, w_next)


def _tc_final(h, s_rows, tp, dinv, b2, ls2_, lb2_):
    def body(h_ref, s_ref, tp_ref, dv_ref, b_ref, ls_ref, lb_ref, ho_ref):
        ho_ref[...] = _layer_math(h_ref[...], s_ref[...], tp_ref[...],
                                  dv_ref[...], b_ref[...], ls_ref[...],
                                  lb_ref[...])

    bs_row = pl.BlockSpec((RB, D), lambda i: (i, 0))
    bs_b = pl.BlockSpec((1, D), lambda i: (0, 0))
    bs_dv = pl.BlockSpec((RB, 1), lambda i: (i, 0))
    return pl.pallas_call(
        body,
        grid=(N // RB,),
        in_specs=[bs_row, bs_row, bs_row, bs_dv, bs_b, bs_b, bs_b],
        out_specs=bs_row,
        out_shape=jax.ShapeDtypeStruct((N, D), jnp.float32),
    )(h, s_rows, tp, dinv, b2, ls2_, lb2_)


def kernel(x, edge_index, W_in, b_in,
           W0, b0, ls0, lb0, W1, b1, ls1, lb1, W2, b2, ls2, lb2):
    src = edge_index[0]
    dst = edge_index[1]
    pad = EPAD - E
    srcp = jnp.concatenate([src, jnp.zeros((pad,), jnp.int32)])
    dstp = jnp.concatenate([dst, jnp.full((pad,), N, jnp.int32)])
    src3 = srcp.reshape(NS, CHUNKS, K)
    dst3 = dstp.reshape(NS, CHUNKS, K)

    degp = _sc_degree(dst3)
    dinv_full = _tc_dinv(degp)
    dinv = dinv_full[:N, 0:1]

    h, tp = _tc_init(x, W_in, b_in.reshape(1, D), W0, dinv)

    for (b, ls_, lb_, w_next) in ((b0, ls0, lb0, W1), (b1, ls1, lb1, W2)):
        s_rows = _sc_scatter(src3, dst3, tp).reshape(NPAD, D)
        h, tp = _tc_layer(h, s_rows, tp, dinv, b.reshape(1, D),
                          ls_.reshape(1, D), lb_.reshape(1, D), w_next)

    s_rows = _sc_scatter(src3, dst3, tp).reshape(NPAD, D)
    return _tc_final(h, s_rows, tp, dinv, b2.reshape(1, D),
                     ls2.reshape(1, D), lb2.reshape(1, D))


# speedup vs baseline: 4.7357x; 1.0010x over previous
import functools

import jax
import jax.numpy as jnp
from jax import lax
from jax.experimental import pallas as pl
from jax.experimental.pallas import tpu as pltpu
from jax.experimental.pallas import tpu_sc as plsc

N = 10000
E = 320000
D = 128
NC = 2
NS = 16
K = 128
CHUNKS = 160
EPT = CHUNKS * K
EPAD = NS * EPT
NPAD = 10240
HALF = NPAD // NC
ACC_R = 5248
DUMP = 5184
DEGW = 16
DRPT = NPAD // NS
ARPT = ACC_R // NS
ORPT = HALF // NS
RB = 1000


def _mesh():
    return plsc.VectorSubcoreMesh(core_axis_name="c", subcore_axis_name="s")


def _sc_degree(dst3):
    HC = CHUNKS // 2

    @functools.partial(
        pl.kernel,
        out_type=jax.ShapeDtypeStruct((NC, NPAD, DEGW), jnp.float32),
        mesh=_mesh(),
        scratch_types=[
            pltpu.VMEM((HC, K), jnp.int32),
            pltpu.VMEM((K, DEGW), jnp.float32),
            pltpu.VMEM((K, DEGW), jnp.float32),
            pltpu.VMEM_SHARED((NPAD, DEGW), jnp.float32),
        ],
    )
    def deg_kernel(dst_hbm, out_hbm, dst_v, ones_v, zero_v, acc):
        c = lax.axis_index("c")
        s = lax.axis_index("s")
        ones16 = jnp.ones((16,), jnp.float32)
        zeros16 = jnp.zeros((16,), jnp.float32)

        def fill(i, carry):
            ones_v[i, pl.ds(0, 16)] = ones16
            zero_v[i, pl.ds(0, 16)] = zeros16
            return carry

        lax.fori_loop(0, K, fill, 0)
        for m in range(DRPT // K):
            pltpu.sync_copy(zero_v, acc.at[pl.ds(s * DRPT + m * K, K)])
        pltpu.sync_copy(dst_hbm.at[s].at[pl.ds(c * HC, HC)], dst_v)
        plsc.subcore_barrier()

        def chunk(j, carry):
            pltpu.sync_copy(ones_v, acc.at[dst_v.at[j]], add=True)
            return carry

        lax.fori_loop(0, HC, chunk, 0)
        plsc.subcore_barrier()
        pltpu.sync_copy(acc.at[pl.ds(s * DRPT, DRPT)],
                        out_hbm.at[c].at[pl.ds(s * DRPT, DRPT)])

    return deg_kernel(dst3)


def _sc_scatter(src3, dst3, tp):
    @functools.partial(
        pl.kernel,
        out_type=jax.ShapeDtypeStruct((NC, HALF, D), jnp.float32),
        mesh=_mesh(),
        scratch_types=[
            pltpu.VMEM((CHUNKS, K), jnp.int32),
            pltpu.VMEM((CHUNKS, K), jnp.int32),
            pltpu.VMEM((K, D), jnp.float32),
            pltpu.VMEM((K, D), jnp.float32),
            pltpu.SemaphoreType.DMA,
            pltpu.VMEM_SHARED((ACC_R, D), jnp.float32),
        ],
    )
    def scat_kernel(src_hbm, dst_hbm, tp_hbm, out_hbm,
                    src_v, dst_v, b0, b1, ssem, acc):
        c = lax.axis_index("c")
        s = lax.axis_index("s")
        zeros16 = jnp.zeros((16,), jnp.float32)

        def fill(i, carry):
            for kk in range(D // 16):
                b0[i, pl.ds(kk * 16, 16)] = zeros16
            return carry

        lax.fori_loop(0, K, fill, 0)
        base = s * ARPT
        for m in range(ARPT // K):
            pltpu.sync_copy(b0, acc.at[pl.ds(base + m * K, K)])
        rem = ARPT % K
        if rem:
            pltpu.sync_copy(b0.at[pl.ds(0, rem)],
                            acc.at[pl.ds(base + (ARPT // K) * K, rem)])
        pltpu.sync_copy(src_hbm.at[s], src_v)
        pltpu.sync_copy(dst_hbm.at[s], dst_v)

        lo_bound = c * HALF

        def remap(j, carry):
            for kk in range(K // 16):
                idx = dst_v[j, pl.ds(kk * 16, 16)]
                loc = idx - lo_bound
                inr = jnp.logical_and(loc >= 0, loc < HALF)
                dump = jnp.full((16,), DUMP, jnp.int32)
                dst_v[j, pl.ds(kk * 16, 16)] = jnp.where(inr, loc, dump)
            return carry

        lax.fori_loop(0, CHUNKS, remap, 0)
        plsc.subcore_barrier()

        def sync_g(j, buf):
            pltpu.sync_copy(tp_hbm.at[src_v.at[j]], buf)

        def step(i, carry):
            j = 2 * i
            sync_g(j, b0)
            d0 = pltpu.async_copy(b0, acc.at[dst_v.at[j]], ssem, add=True)
            sync_g(j + 1, b1)
            d1 = pltpu.async_copy(b1, acc.at[dst_v.at[j + 1]], ssem, add=True)
            d0.wait()
            d1.wait()
            return carry

        lax.fori_loop(0, CHUNKS // 2, step, 0)
        plsc.subcore_barrier()
        pltpu.sync_copy(acc.at[pl.ds(s * ORPT, ORPT)],
                        out_hbm.at[c].at[pl.ds(s * ORPT, ORPT)])

    return scat_kernel(src3, dst3, tp)


def _tc_dinv(degp):
    def body(p_ref, o_ref):
        p = p_ref[...]
        o_ref[...] = lax.rsqrt(1.0 + p[0] + p[1])

    return pl.pallas_call(
        body, out_shape=jax.ShapeDtypeStruct((NPAD, DEGW), jnp.float32)
    )(degp)


_DN = (((1,), (1,)), ((), ()))


def _tc_init(x, w_in, b_in2, w0, dinv):
    def body(x_ref, wi_ref, bi_ref, w0_ref, dv_ref, h_ref, tp_ref):
        h = lax.dot_general(x_ref[...], wi_ref[...], _DN,
                            preferred_element_type=jnp.float32) + bi_ref[...]
        h_ref[...] = h
        t = lax.dot_general(h, w0_ref[...], _DN,
                            preferred_element_type=jnp.float32)
        tp_ref[...] = t * dv_ref[...]

    bs_row = pl.BlockSpec((RB, D), lambda i: (i, 0))
    bs_w = pl.BlockSpec((D, D), lambda i: (0, 0))
    bs_b = pl.BlockSpec((1, D), lambda i: (0, 0))
    bs_dv = pl.BlockSpec((RB, 1), lambda i: (i, 0))
    return pl.pallas_call(
        body,
        grid=(N // RB,),
        in_specs=[bs_row, bs_w, bs_b, bs_w, bs_dv],
        out_specs=[bs_row, bs_row],
        out_shape=[jax.ShapeDtypeStruct((N, D), jnp.float32)] * 2,
    )(x, w_in, b_in2, w0, dinv)


def _layer_math(h, s_rows, tp, dv, b, ls_, lb_):
    agg = dv * (s_rows + tp) + b
    hn = jnp.where(agg > 0, agg, jnp.exp(agg) - 1.0)
    r = hn + h
    mu = jnp.mean(r, axis=-1, keepdims=True)
    dd = r - mu
    var = jnp.mean(dd * dd, axis=-1, keepdims=True)
    return dd * lax.rsqrt(var + 1e-5) * ls_ + lb_


def _tc_layer(h, s_rows, tp, dinv, b2, ls2_, lb2_, w_next):
    def body(h_ref, s_ref, tp_ref, dv_ref, b_ref, ls_ref, lb_ref, wn_ref,
             ho_ref, tpn_ref):
        dv = dv_ref[...]
        y = _layer_math(h_ref[...], s_ref[...], tp_ref[...], dv,
                        b_ref[...], ls_ref[...], lb_ref[...])
        ho_ref[...] = y
        tpn_ref[...] = lax.dot_general(y, wn_ref[...], _DN,
                                       preferred_element_type=jnp.float32) * dv

    bs_row = pl.BlockSpec((RB, D), lambda i: (i, 0))
    bs_w = pl.BlockSpec((D, D), lambda i: (0, 0))
    bs_b = pl.BlockSpec((1, D), lambda i: (0, 0))
    bs_dv = pl.BlockSpec((RB, 1), lambda i: (i, 0))
    return pl.pallas_call(
        body,
        grid=(N // RB,),
        in_specs=[bs_row, bs_row, bs_row, bs_dv, bs_b, bs_b, bs_b, bs_w],
        out_specs=[bs_row, bs_row],
        out_shape=[jax.ShapeDtypeStruct((N, D), jnp.float32)] * 2,
    )(h, s_rows, tp, dinv, b2, ls2_, lb2_, w_next)


def _tc_final(h, s_rows, tp, dinv, b2, ls2_, lb2_):
    def body(h_ref, s_ref, tp_ref, dv_ref, b_ref, ls_ref, lb_ref, ho_ref):
        ho_ref[...] = _layer_math(h_ref[...], s_ref[...], tp_ref[...],
                                  dv_ref[...], b_ref[...], ls_ref[...],
                                  lb_ref[...])

    bs_row = pl.BlockSpec((RB, D), lambda i: (i, 0))
    bs_b = pl.BlockSpec((1, D), lambda i: (0, 0))
    bs_dv = pl.BlockSpec((RB, 1), lambda i: (i, 0))
    return pl.pallas_call(
        body,
        grid=(N // RB,),
        in_specs=[bs_row, bs_row, bs_row, bs_dv, bs_b, bs_b, bs_b],
        out_specs=bs_row,
        out_shape=jax.ShapeDtypeStruct((N, D), jnp.float32),
    )(h, s_rows, tp, dinv, b2, ls2_, lb2_)


def kernel(x, edge_index, W_in, b_in,
           W0, b0, ls0, lb0, W1, b1, ls1, lb1, W2, b2, ls2, lb2):
    src = edge_index[0]
    dst = edge_index[1]
    pad = EPAD - E
    srcp = jnp.concatenate([src, jnp.zeros((pad,), jnp.int32)])
    dstp = jnp.concatenate([dst, jnp.full((pad,), N, jnp.int32)])
    src3 = srcp.reshape(NS, CHUNKS, K)
    dst3 = dstp.reshape(NS, CHUNKS, K)

    degp = _sc_degree(dst3)
    dinv_full = _tc_dinv(degp)
    dinv = dinv_full[:N, 0:1]

    h, tp = _tc_init(x, W_in, b_in.reshape(1, D), W0, dinv)

    for (b, ls_, lb_, w_next) in ((b0, ls0, lb0, W1), (b1, ls1, lb1, W2)):
        s_rows = _sc_scatter(src3, dst3, tp).reshape(NPAD, D)
        h, tp = _tc_layer(h, s_rows, tp, dinv, b.reshape(1, D),
                          ls_.reshape(1, D), lb_.reshape(1, D), w_next)

    s_rows = _sc_scatter(src3, dst3, tp).reshape(NPAD, D)
    return _tc_final(h, s_rows, tp, dinv, b2.reshape(1, D),
                     ls2.reshape(1, D), lb2.reshape(1, D))
